# baseline (speedup 1.0000x reference)
import dataclasses
import functools

import jax
import jax.numpy as jnp
import numpy as np
from jax import lax
from jax.experimental import pallas as pl
from jax.experimental.pallas import tpu as pltpu
from jax.experimental.pallas import tpu_sc as plsc

B, S, V, D = 4096, 200, 1000000, 64
NC, NS = 2, 16
NW = NC * NS
BB = 128
NBLK = S * (B // BB)
BLK_PER_W = NBLK // NW
BPS = B // BB
LANES = 16
NSLOT = 4
WPAD = 133


def _positional_encoding() -> np.ndarray:
    pos = np.arange(S, dtype=np.float32)[:, None]
    i = np.arange(D, dtype=np.float32)[None, :]
    angle_rates = 1.0 / np.power(10000.0, (2.0 * np.floor(i / 2.0)) / np.float32(D))
    angle_rads = pos * angle_rates
    pe = np.zeros((S, D), dtype=np.float32)
    pe[:, 0::2] = np.sin(angle_rads[:, 0::2])
    pe[:, 1::2] = np.cos(angle_rads[:, 1::2])
    return pe


_PE = _positional_encoding()


def _sc_compiler_params():
    cp = pltpu.CompilerParams(use_tc_tiling_on_sc=False)
    if "needs_layout_passes" in pltpu.CompilerParams.__dataclass_fields__:
        cp = dataclasses.replace(cp, needs_layout_passes=False)
    return cp


def kernel(inputs, table):
    idx_tiles = (inputs.T.reshape(S // 8, 8, B // BB, BB)
                 .transpose(0, 2, 1, 3).reshape(S * B))
    pe = jnp.asarray(_PE)

    mesh = plsc.VectorSubcoreMesh(core_axis_name="c", subcore_axis_name="s")

    @functools.partial(
        pl.kernel,
        out_type=jax.ShapeDtypeStruct((S, D // 8, B // BB, 8, BB), jnp.float32),
        mesh=mesh,
        compiler_params=_sc_compiler_params(),
        scratch_types=[
            pltpu.VMEM((BLK_PER_W * BB,), jnp.int32),
            pltpu.VMEM((S, D), jnp.float32),
        ]
        + [pltpu.VMEM((BB, D), jnp.float32) for _ in range(NSLOT)]
        + [pltpu.VMEM((D // 8, 8, WPAD), jnp.float32) for _ in range(NSLOT)]
        + [pltpu.SemaphoreType.DMA for _ in range(2 * NSLOT)],
    )
    def run(idx_hbm, table_hbm, pe_hbm, out_hbm, idx_v, pe_v, *bufs):
        rows = bufs[:NSLOT]
        wblk = bufs[NSLOT:2 * NSLOT]
        gsem = bufs[2 * NSLOT:3 * NSLOT]
        wsem = bufs[3 * NSLOT:4 * NSLOT]

        wid = lax.axis_index("s") * NC + lax.axis_index("c")
        gbase = wid * BLK_PER_W
        pltpu.sync_copy(idx_hbm.at[pl.ds(gbase * BB, BLK_PER_W * BB)], idx_v)
        pltpu.sync_copy(pe_hbm, pe_v)

        lane = jnp.arange(LANES, dtype=jnp.int32)
        din_idx = lane % 8
        dt_base = lane // 8

        def seq_bt(j):
            h = gbase + j
            s = 8 * (h // 256) + lax.rem(h, 8)
            bt = lax.rem(h // 8, BPS)
            return s, bt

        def gather(j, p):
            return pltpu.make_async_copy(
                table_hbm.at[idx_v.at[pl.ds(j * BB, BB)]], rows[p], gsem[p])

        def wb(j, p):
            s, bt = seq_bt(j)
            return pltpu.make_async_copy(
                wblk[p].at[:, :, pl.ds(0, BB)], out_hbm.at[s, :, bt], wsem[p])

        def compute(j, p):
            s, _ = seq_bt(j)
            pe_vecs = [pe_v[s, pl.ds(g * LANES, LANES)] for g in range(D // LANES)]
            dt_vecs = [dt_base + 2 * g for g in range(D // LANES)]

            @pl.loop(0, BB, step=4)
            def _tok(t0):
                for tt in range(4):
                    t = t0 + tt
                    t_splat = jnp.full((LANES,), 0, dtype=jnp.int32) + t
                    for g in range(D // LANES):
                        v = rows[p][t, pl.ds(g * LANES, LANES)] + pe_vecs[g]
                        plsc.store_scatter(
                            wblk[p], [dt_vecs[g], din_idx, t_splat], v)

        for k in range(NSLOT):
            gather(k, k).start()
        for k in range(NSLOT):
            gather(k, k).wait()
            compute(k, k)
            wb(k, k).start()
            gather(k + NSLOT, k).start()

        @pl.loop(NSLOT, BLK_PER_W, step=NSLOT)
        def _body(j):
            for k in range(NSLOT):
                gather(j + k, k).wait()
                wb(j + k - NSLOT, k).wait()
                compute(j + k, k)
                wb(j + k, k).start()

                @pl.when(j + k + NSLOT < BLK_PER_W)
                def _():
                    gather(j + k + NSLOT, k).start()

        for k in range(NSLOT):
            wb(BLK_PER_W - NSLOT + k, k).wait()

    out5d = run(idx_tiles, table, pe)
    return out5d.transpose(2, 4, 0, 1, 3).reshape(B, S, D)
